# baseline (speedup 1.0000x reference)
import functools

import jax
import jax.numpy as jnp
from jax import lax
from jax.experimental import pallas as pl
from jax.experimental.pallas import tpu as pltpu
from jax.experimental.pallas import tpu_sc as plsc

MAX_POSITION = 2048
DEPTH = 64


@functools.partial(jax.jit, static_argnums=(1,))
def _rpe_expand(table, length):
    L = length
    D = table.shape[-1]
    info = plsc.get_sparse_core_info()
    nc, ns = info.num_cores, info.num_subcores
    nw = nc * ns
    ppw = L // nw
    win = 2 * L - nw
    nst = D // 8
    nlt = L // 128

    M = MAX_POSITION
    F = jnp.flip(table[M - L:M + L], axis=0).T
    F8 = jnp.stack([jnp.pad(F, ((0, 0), (r, 8 - r))) for r in range(8)])
    F8 = F8.reshape(8, nst, 8, 2 * L + 8)

    mesh = plsc.VectorSubcoreMesh(core_axis_name="c", subcore_axis_name="s")

    @functools.partial(
        pl.kernel,
        mesh=mesh,
        out_type=jax.ShapeDtypeStruct((L, nst, nlt, 8, 128), jnp.float32),
        scratch_types=[
            pltpu.VMEM((nst, 8, win), jnp.float32),
            pltpu.SemaphoreType.DMA,
        ],
        compiler_params=pltpu.CompilerParams(use_tc_tiling_on_sc=False),
    )
    def k(f8_hbm, out_hbm, buf, sem):
        wid = lax.axis_index("s") * nc + lax.axis_index("c")
        r = lax.rem(wid, 8)
        a0 = pl.multiple_of(nw - wid + r, 8)
        pltpu.sync_copy(f8_hbm.at[r, :, :, pl.ds(a0, win)], buf)

        def plane_body(m, carry):
            i = wid + nw * m
            s = pl.multiple_of(L - nw - nw * m, 8)
            copies = [
                pltpu.async_copy(
                    buf.at[:, :, pl.ds(s + 128 * lt, 128)],
                    out_hbm.at[i, :, lt],
                    sem,
                )
                for lt in range(nlt)
            ]
            for c in copies:
                c.wait()
            return carry

        lax.fori_loop(0, ppw, plane_body, 0)

    q = k(F8)
    return jnp.transpose(q, (0, 2, 4, 1, 3)).reshape(L, L, D)


def kernel(inputs, rel_embeddings):
    return _rpe_expand(rel_embeddings, inputs.shape[1])


# speedup vs baseline: 1.0306x; 1.0306x over previous
import functools

import jax
import jax.numpy as jnp
from jax import lax
from jax.experimental import pallas as pl
from jax.experimental.pallas import tpu as pltpu
from jax.experimental.pallas import tpu_sc as plsc

MAX_POSITION = 2048
DEPTH = 64


@functools.partial(jax.jit, static_argnums=(1,))
def _rpe_expand(table, length):
    L = length
    D = table.shape[-1]
    info = plsc.get_sparse_core_info()
    nc, ns = info.num_cores, info.num_subcores
    nw = nc * ns
    ppw = L // nw
    win = 2 * L - nw
    nst = D // 8
    nlt = L // 128

    M = MAX_POSITION
    F = jnp.flip(table[M - L:M + L], axis=0).T
    F8 = jnp.stack([jnp.pad(F, ((0, 0), (r, 8 - r))) for r in range(8)])

    mesh = plsc.VectorSubcoreMesh(core_axis_name="c", subcore_axis_name="s")

    @functools.partial(
        pl.kernel,
        mesh=mesh,
        out_type=jax.ShapeDtypeStruct((L, nst, nlt, 8, 128), jnp.float32),
        scratch_types=[
            pltpu.VMEM((D, win), jnp.float32),
            pltpu.SemaphoreType.DMA,
        ],
        compiler_params=pltpu.CompilerParams(use_tc_tiling_on_sc=False),
    )
    def k(f8_hbm, out_hbm, buf, sem):
        wid = lax.axis_index("s") * nc + lax.axis_index("c")
        r = lax.rem(wid, 8)
        a0 = pl.multiple_of(nw - wid + r, 8)
        pltpu.sync_copy(f8_hbm.at[r, :, pl.ds(a0, win)], buf)

        def plane_body(m, carry):
            i = wid + nw * m
            s = pl.multiple_of(L - nw - nw * m, 8)
            copies = [
                pltpu.async_copy(
                    buf.at[pl.ds(8 * st, 8), pl.ds(s + 128 * lt, 128)],
                    out_hbm.at[i, st, lt],
                    sem,
                )
                for st in range(nst)
                for lt in range(nlt)
            ]
            for c in copies:
                c.wait()
            return carry

        lax.fori_loop(0, ppw, plane_body, 0)

    q = k(F8)
    return jnp.transpose(q, (0, 2, 4, 1, 3)).reshape(L, L, D)


def kernel(inputs, rel_embeddings):
    return _rpe_expand(rel_embeddings, inputs.shape[1])
